# baseline (speedup 1.0000x reference)
import functools

import jax
import jax.numpy as jnp
from jax import lax
from jax.experimental import pallas as pl
from jax.experimental.pallas import tpu as pltpu
from jax.experimental.pallas import tpu_sc as plsc

B, L, D = 4, 4096, 2048
NW = 32
ROWS = B * L
RPW = ROWS // NW
R = 8
CHUNK = R * D
NCHUNK = RPW // R

_mesh = plsc.VectorSubcoreMesh(core_axis_name="c", subcore_axis_name="s")

_vmem = lambda: pltpu.VMEM((CHUNK,), jnp.float32)


@functools.partial(
    pl.kernel,
    out_type=jax.ShapeDtypeStruct((ROWS * D,), jnp.float32),
    mesh=_mesh,
    scratch_types=[
        _vmem(), _vmem(), _vmem(),
        _vmem(), _vmem(), _vmem(),
        pltpu.SemaphoreType.DMA, pltpu.SemaphoreType.DMA,
        pltpu.SemaphoreType.DMA, pltpu.SemaphoreType.DMA,
        pltpu.SemaphoreType.DMA, pltpu.SemaphoreType.DMA,
    ],
)
def _sc_add(x_hbm, emb_hbm, out_hbm, xv0, ev0, ov0, xv1, ev1, ov1,
            sx0, se0, so0, sx1, se1, so1):
    wid = lax.axis_index("s") * 2 + lax.axis_index("c")
    base = wid * RPW
    ebase = lax.rem(base, L)
    bufs = ((xv0, ev0, ov0, sx0, se0, so0), (xv1, ev1, ov1, sx1, se1, so1))

    def xoff(i):
        return (base + i * R) * D

    def eoff(i):
        return (ebase + i * R) * D

    for b in range(2):
        xv, ev, _, sx, se, _ = bufs[b]
        pltpu.async_copy(x_hbm.at[pl.ds(xoff(b), CHUNK)], xv, sx)
        pltpu.async_copy(emb_hbm.at[pl.ds(eoff(b), CHUNK)], ev, se)

    def round_body(k, _):
        for b in range(2):
            i = 2 * k + b
            xv, ev, ov, sx, se, so = bufs[b]
            pltpu.make_async_copy(x_hbm.at[pl.ds(xoff(i), CHUNK)], xv, sx).wait()
            pltpu.make_async_copy(emb_hbm.at[pl.ds(eoff(i), CHUNK)], ev, se).wait()
            @pl.when(k > 0)
            def _wait_store():
                pltpu.make_async_copy(
                    ov, out_hbm.at[pl.ds(xoff(i - 2), CHUNK)], so
                ).wait()

            @plsc.parallel_loop(0, CHUNK // 16, unroll=8)
            def _vec_body(j):
                o = j * 16
                ov[pl.ds(o, 16)] = xv[pl.ds(o, 16)] + ev[pl.ds(o, 16)]
            pltpu.async_copy(ov, out_hbm.at[pl.ds(xoff(i), CHUNK)], so)

            @pl.when(i + 2 < NCHUNK)
            def _prefetch():
                pltpu.async_copy(x_hbm.at[pl.ds(xoff(i + 2), CHUNK)], xv, sx)
                pltpu.async_copy(emb_hbm.at[pl.ds(eoff(i + 2), CHUNK)], ev, se)

        return 0

    lax.fori_loop(0, NCHUNK // 2, round_body, 0)
    for b in range(2):
        _, _, ov, _, _, so = bufs[b]
        i = NCHUNK - 2 + b
        pltpu.make_async_copy(ov, out_hbm.at[pl.ds(xoff(i), CHUNK)], so).wait()


def kernel(x, emb_weight):
    out = _sc_add(x.reshape(ROWS * D), emb_weight.reshape(L * D))
    return out.reshape(B, L, D)


# speedup vs baseline: 1.0027x; 1.0027x over previous
import functools

import jax
import jax.numpy as jnp
from jax import lax
from jax.experimental import pallas as pl
from jax.experimental.pallas import tpu as pltpu
from jax.experimental.pallas import tpu_sc as plsc

B, L, D = 4, 4096, 2048
NW = 32
ROWS = B * L
RPW = ROWS // NW
R = 8
CHUNK = R * D
NCHUNK = RPW // R

_mesh = plsc.VectorSubcoreMesh(core_axis_name="c", subcore_axis_name="s")

_vmem = lambda: pltpu.VMEM((CHUNK,), jnp.float32)


@functools.partial(
    pl.kernel,
    out_type=jax.ShapeDtypeStruct((ROWS * D,), jnp.float32),
    mesh=_mesh,
    scratch_types=[
        _vmem(), _vmem(), _vmem(),
        _vmem(), _vmem(), _vmem(),
        pltpu.SemaphoreType.DMA, pltpu.SemaphoreType.DMA,
        pltpu.SemaphoreType.DMA, pltpu.SemaphoreType.DMA,
        pltpu.SemaphoreType.DMA, pltpu.SemaphoreType.DMA,
    ],
)
def _sc_add(x_hbm, emb_hbm, out_hbm, xv0, ev0, ov0, xv1, ev1, ov1,
            sx0, se0, so0, sx1, se1, so1):
    wid = lax.axis_index("s") * 2 + lax.axis_index("c")
    base = wid * RPW
    ebase = lax.rem(base, L)
    bufs = ((xv0, ev0, ov0, sx0, se0, so0), (xv1, ev1, ov1, sx1, se1, so1))

    def xoff(i):
        return (base + i * R) * D

    def eoff(i):
        return (ebase + i * R) * D

    for b in range(2):
        xv, ev, _, sx, se, _ = bufs[b]
        pltpu.async_copy(x_hbm.at[pl.ds(xoff(b), CHUNK)], xv, sx)
        pltpu.async_copy(emb_hbm.at[pl.ds(eoff(b), CHUNK)], ev, se)

    def round_body(k, _):
        for b in range(2):
            i = 2 * k + b
            xv, ev, ov, sx, se, so = bufs[b]
            pltpu.make_async_copy(x_hbm.at[pl.ds(xoff(i), CHUNK)], xv, sx).wait()
            pltpu.make_async_copy(emb_hbm.at[pl.ds(eoff(i), CHUNK)], ev, se).wait()
            @pl.when(k > 0)
            def _wait_store():
                pltpu.make_async_copy(
                    ov, out_hbm.at[pl.ds(xoff(i - 2), CHUNK)], so
                ).wait()

            @plsc.parallel_loop(0, CHUNK // 16, unroll=16)
            def _vec_body(j):
                o = j * 16
                ov[pl.ds(o, 16)] = xv[pl.ds(o, 16)] + ev[pl.ds(o, 16)]
            pltpu.async_copy(ov, out_hbm.at[pl.ds(xoff(i), CHUNK)], so)

            @pl.when(i + 2 < NCHUNK)
            def _prefetch():
                pltpu.async_copy(x_hbm.at[pl.ds(xoff(i + 2), CHUNK)], xv, sx)
                pltpu.async_copy(emb_hbm.at[pl.ds(eoff(i + 2), CHUNK)], ev, se)

        return 0

    lax.fori_loop(0, NCHUNK // 2, round_body, 0)
    for b in range(2):
        _, _, ov, _, _, so = bufs[b]
        i = NCHUNK - 2 + b
        pltpu.make_async_copy(ov, out_hbm.at[pl.ds(xoff(i), CHUNK)], so).wait()


def kernel(x, emb_weight):
    out = _sc_add(x.reshape(ROWS * D), emb_weight.reshape(L * D))
    return out.reshape(B, L, D)


# speedup vs baseline: 1.0835x; 1.0806x over previous
import jax
import jax.numpy as jnp
from jax import lax
from jax.experimental import pallas as pl
from jax.experimental.pallas import tpu as pltpu
from jax.experimental.pallas import tpu_sc as plsc

B, L, D = 4, 4096, 2048
NW = 32
LPW = L // NW
R = 8
CHUNK = R * D
NCH = LPW // R
VECS = CHUNK // 16

_mesh = plsc.VectorSubcoreMesh(core_axis_name="c", subcore_axis_name="s")

_vmem = lambda: pltpu.VMEM((CHUNK,), jnp.float32)


def _kernel_fn(x_hbm, emb_hbm, out_hbm, ev0, ev1, xv0, xv1, ov0, ov1,
               sev0, sev1, sxv0, sxv1, sov0, sov1):
    wid = lax.axis_index("s") * 2 + lax.axis_index("c")
    lbase = wid * LPW
    evb = ((ev0, sev0), (ev1, sev1))
    xvb = ((xv0, sxv0), (xv1, sxv1))
    ovb = ((ov0, sov0), (ov1, sov1))

    def eoff(i):
        return (lbase + i * R) * D

    def xoff(i, b):
        return (b * L + lbase + i * R) * D

    def load_e(i, slot):
        ev, se = evb[slot]
        return pltpu.make_async_copy(emb_hbm.at[pl.ds(eoff(i), CHUNK)], ev, se)

    def load_x(i, b, slot):
        xv, sx = xvb[slot]
        return pltpu.make_async_copy(x_hbm.at[pl.ds(xoff(i, b), CHUNK)], xv, sx)

    def store_o(i, b, slot):
        ov, so = ovb[slot]
        return pltpu.make_async_copy(ov, out_hbm.at[pl.ds(xoff(i, b), CHUNK)], so)

    load_e(0, 0).start()
    load_x(0, 0, 0).start()
    load_x(0, 1, 1).start()

    def round_body(k, _):
        for ip in range(2):
            i = 2 * k + ip
            load_e(i, ip).wait()
            for b in range(4):
                s = b % 2
                load_x(i, b, s).wait()
                first = (i == 0) & (b < 2)

                @pl.when(jnp.logical_not(first))
                def _wait_store():
                    pi, pb = (i, b - 2) if b >= 2 else (i - 1, b + 2)
                    store_o(pi, pb, s).wait()

                ev = evb[ip][0]
                xv = xvb[s][0]
                ov = ovb[s][0]

                @plsc.parallel_loop(0, VECS, unroll=8)
                def _vec_body(j):
                    o = j * 16
                    ov[pl.ds(o, 16)] = xv[pl.ds(o, 16)] + ev[pl.ds(o, 16)]

                store_o(i, b, s).start()
                if b < 2:
                    load_x(i, b + 2, s).start()
                else:
                    @pl.when(i + 1 < NCH)
                    def _pf_x():
                        load_x(i + 1, b - 2, s).start()
                if b == 0:
                    @pl.when(i + 1 < NCH)
                    def _pf_e():
                        load_e(i + 1, 1 - ip).start()
        return 0

    lax.fori_loop(0, NCH // 2, round_body, 0)
    store_o(NCH - 1, 2, 0).wait()
    store_o(NCH - 1, 3, 1).wait()


_sc_add = pl.kernel(
    _kernel_fn,
    out_type=jax.ShapeDtypeStruct((B * L * D,), jnp.float32),
    mesh=_mesh,
    scratch_types=[
        _vmem(), _vmem(),
        _vmem(), _vmem(),
        _vmem(), _vmem(),
        pltpu.SemaphoreType.DMA, pltpu.SemaphoreType.DMA,
        pltpu.SemaphoreType.DMA, pltpu.SemaphoreType.DMA,
        pltpu.SemaphoreType.DMA, pltpu.SemaphoreType.DMA,
    ],
)


def kernel(x, emb_weight):
    out = _sc_add(x.reshape(B * L * D), emb_weight.reshape(L * D))
    return out.reshape(B, L, D)


# speedup vs baseline: 4.5149x; 4.1672x over previous
import jax
import jax.numpy as jnp
from jax.experimental import pallas as pl

B, L, D = 4, 4096, 2048
BL = 2048
BD = 1024


def _add_kernel(x_ref, emb_ref, o_ref):
    o_ref[...] = x_ref[...] + emb_ref[...][None, :, :]


def kernel(x, emb_weight):
    nl = L // BL
    nd = D // BD
    return pl.pallas_call(
        _add_kernel,
        grid=(nl, nd, B),
        in_specs=[
            pl.BlockSpec((1, BL, BD), lambda l, d, b: (b, l, d)),
            pl.BlockSpec((BL, BD), lambda l, d, b: (l, d)),
        ],
        out_specs=pl.BlockSpec((1, BL, BD), lambda l, d, b: (b, l, d)),
        out_shape=jax.ShapeDtypeStruct((B, L, D), x.dtype),
    )(x, emb_weight)
